# baseline (speedup 1.0000x reference)
import functools

import jax
import jax.numpy as jnp
from jax.experimental import pallas as pl
from jax.experimental.pallas import tpu as pltpu

NUM_EXPERTS = 64
TOP_K = 2
HIDDEN = 2048
Z_LOSS_COEFF = 0.001
AUX_LOSS_COEFF = 0.01

BLOCK_T = 256
NSTREAM = 2


def _router_rows(logits):
    rowmax = jnp.max(logits, axis=1, keepdims=True)
    ex = jnp.exp(logits - rowmax)
    denom = jnp.sum(ex, axis=1, keepdims=True)
    scores = ex / denom
    lse = rowmax + jnp.log(denom)

    iota = jax.lax.broadcasted_iota(jnp.int32, scores.shape, 1)
    m1 = jnp.max(scores, axis=1, keepdims=True)
    idx1 = jnp.min(jnp.where(scores == m1, iota, NUM_EXPERTS),
                   axis=1, keepdims=True)
    masked = jnp.where(iota == idx1, -1.0, scores)
    m2 = jnp.max(masked, axis=1, keepdims=True)
    idx2 = jnp.min(jnp.where(masked == m2, iota, NUM_EXPERTS),
                   axis=1, keepdims=True)

    wts = jnp.concatenate([m1, m2], axis=1)
    idx = jnp.concatenate([idx1, idx2], axis=1)

    onehot = ((iota == idx1) | (iota == idx2)).astype(jnp.float32)
    hist_part = jnp.sum(onehot, axis=0, keepdims=True)
    agg_part = jnp.sum(scores, axis=0, keepdims=True)
    z_part = jnp.sum(lse * lse)
    return wts, idx, hist_part, agg_part, z_part


def _router_block(*refs, nblk, num_tokens):
    x_refs = refs[:NSTREAM]
    w_ref = refs[NSTREAM]
    wts_refs = refs[NSTREAM + 1:2 * NSTREAM + 1]
    idx_refs = refs[2 * NSTREAM + 1:3 * NSTREAM + 1]
    z_ref = refs[3 * NSTREAM + 1]
    lbl_ref = refs[3 * NSTREAM + 2]
    agg_ref, hist_ref, zacc_ref = refs[3 * NSTREAM + 3:]

    i = pl.program_id(0)
    w = w_ref[...]

    hist_acc = jnp.zeros((1, NUM_EXPERTS), jnp.float32)
    agg_acc = jnp.zeros((1, NUM_EXPERTS), jnp.float32)
    z_acc = jnp.float32(0.0)
    for x_ref, wts_ref, idx_ref in zip(x_refs, wts_refs, idx_refs):
        logits = jax.lax.dot_general(
            x_ref[...], w,
            dimension_numbers=(((1,), (1,)), ((), ())),
            preferred_element_type=jnp.float32,
        )
        wts, idx, hist_part, agg_part, z_part = _router_rows(logits)
        wts_ref[...] = wts
        idx_ref[...] = idx
        hist_acc += hist_part
        agg_acc += agg_part
        z_acc += z_part

    @pl.when(i == 0)
    def _init():
        agg_ref[...] = agg_acc
        hist_ref[...] = hist_acc
        zacc_ref[0, 0] = z_acc

    @pl.when(i > 0)
    def _accum():
        agg_ref[...] += agg_acc
        hist_ref[...] += hist_acc
        zacc_ref[0, 0] += z_acc

    @pl.when(i == nblk - 1)
    def _finalize():
        z_ref[...] = jnp.reshape(zacc_ref[0, 0] / num_tokens * Z_LOSS_COEFF,
                                 (1, 1))
        lbl = jnp.sum(agg_ref[...] * hist_ref[...], keepdims=True) * (
            NUM_EXPERTS * AUX_LOSS_COEFF / (num_tokens * num_tokens * TOP_K))
        lbl_ref[...] = jnp.reshape(lbl, (1, 1))


def kernel(x, W):
    xf = x.reshape(-1, x.shape[-1])
    num_tokens = xf.shape[0]
    nblk = num_tokens // (BLOCK_T * NSTREAM)
    stream_rows = num_tokens // NSTREAM

    def x_map(s):
        return lambda i: (i + s * nblk, 0)

    in_specs = [pl.BlockSpec((BLOCK_T, HIDDEN), x_map(s))
                for s in range(NSTREAM)]
    in_specs.append(pl.BlockSpec((NUM_EXPERTS, HIDDEN), lambda i: (0, 0)))

    out_specs = (
        [pl.BlockSpec((BLOCK_T, TOP_K), lambda i: (i, 0))
         for _ in range(2 * NSTREAM)]
        + [pl.BlockSpec((1, 1), lambda i: (0, 0))] * 2)
    out_shape = (
        [jax.ShapeDtypeStruct((stream_rows, TOP_K), jnp.float32)
         for _ in range(NSTREAM)]
        + [jax.ShapeDtypeStruct((stream_rows, TOP_K), jnp.int32)
           for _ in range(NSTREAM)]
        + [jax.ShapeDtypeStruct((1, 1), jnp.float32)] * 2)
    out_specs = out_specs[:NSTREAM] + out_specs[:NSTREAM] + out_specs[-2:]

    outs = pl.pallas_call(
        functools.partial(_router_block, nblk=nblk, num_tokens=num_tokens),
        grid=(nblk,),
        in_specs=in_specs,
        out_specs=out_specs,
        out_shape=out_shape,
        scratch_shapes=[
            pltpu.VMEM((1, NUM_EXPERTS), jnp.float32),
            pltpu.VMEM((1, NUM_EXPERTS), jnp.float32),
            pltpu.SMEM((1, 1), jnp.float32),
        ],
    )(*([xf] * NSTREAM), W)

    wts = jnp.concatenate(outs[:NSTREAM], axis=0)
    idx = jnp.concatenate(outs[NSTREAM:2 * NSTREAM], axis=0)
    z, lbl = outs[2 * NSTREAM], outs[2 * NSTREAM + 1]
    return wts, idx, z[0, 0], lbl[0, 0]


# speedup vs baseline: 1.0537x; 1.0537x over previous
import functools

import jax
import jax.numpy as jnp
from jax.experimental import pallas as pl
from jax.experimental.pallas import tpu as pltpu

NUM_EXPERTS = 64
TOP_K = 2
HIDDEN = 2048
Z_LOSS_COEFF = 0.001
AUX_LOSS_COEFF = 0.01

BLOCK_T = 1024
NSTREAM = 2
NEG_HUGE = -3.0e38


def _router_rows(logits):
    iota = jax.lax.broadcasted_iota(jnp.int32, logits.shape, 1)

    m1 = jnp.max(logits, axis=1, keepdims=True)
    idx1 = jnp.min(jnp.where(logits == m1, iota, NUM_EXPERTS),
                   axis=1, keepdims=True)
    masked = jnp.where(iota == idx1, NEG_HUGE, logits)
    m2 = jnp.max(masked, axis=1, keepdims=True)
    idx2 = jnp.min(jnp.where(masked == m2, iota, NUM_EXPERTS),
                   axis=1, keepdims=True)

    ex = jnp.exp(logits - m1)
    denom = jnp.sum(ex, axis=1, keepdims=True)
    recip = 1.0 / denom
    lse = m1 + jnp.log(denom)

    w1 = recip
    w2 = jnp.exp(m2 - m1) * recip

    iota2 = jax.lax.broadcasted_iota(jnp.int32, (logits.shape[0], TOP_K), 1)
    wts = jnp.where(iota2 == 0, w1, w2)
    idx = jnp.where(iota2 == 0, idx1, idx2)

    onehot = ((iota == idx1) | (iota == idx2)).astype(jnp.float32)
    hist_part = jnp.sum(onehot, axis=0, keepdims=True)
    agg_part = jnp.sum(ex * recip, axis=0, keepdims=True)
    z_part = jnp.sum(lse * lse, axis=0, keepdims=True)
    return wts, idx, hist_part, agg_part, z_part


def _router_block(*refs, nblk, num_tokens):
    x_refs = refs[:NSTREAM]
    w_ref = refs[NSTREAM]
    wts_refs = refs[NSTREAM + 1:2 * NSTREAM + 1]
    idx_refs = refs[2 * NSTREAM + 1:3 * NSTREAM + 1]
    z_ref = refs[3 * NSTREAM + 1]
    lbl_ref = refs[3 * NSTREAM + 2]
    agg_ref, hist_ref, zacc_ref = refs[3 * NSTREAM + 3:]

    i = pl.program_id(0)
    w = w_ref[...]

    hist_acc = jnp.zeros((1, NUM_EXPERTS), jnp.float32)
    agg_acc = jnp.zeros((1, NUM_EXPERTS), jnp.float32)
    z_acc = jnp.zeros((1, 1), jnp.float32)
    for x_ref, wts_ref, idx_ref in zip(x_refs, wts_refs, idx_refs):
        logits = jax.lax.dot_general(
            x_ref[...], w,
            dimension_numbers=(((1,), (1,)), ((), ())),
            preferred_element_type=jnp.float32,
        )
        wts, idx, hist_part, agg_part, z_part = _router_rows(logits)
        wts_ref[...] = wts
        idx_ref[...] = idx
        hist_acc += hist_part
        agg_acc += agg_part
        z_acc += z_part

    @pl.when(i == 0)
    def _init():
        agg_ref[...] = agg_acc
        hist_ref[...] = hist_acc
        zacc_ref[...] = z_acc

    @pl.when(i > 0)
    def _accum():
        agg_ref[...] += agg_acc
        hist_ref[...] += hist_acc
        zacc_ref[...] += z_acc

    @pl.when(i == nblk - 1)
    def _finalize():
        z_ref[...] = zacc_ref[...] * (Z_LOSS_COEFF / num_tokens)
        lbl_ref[...] = jnp.sum(agg_ref[...] * hist_ref[...], keepdims=True) * (
            NUM_EXPERTS * AUX_LOSS_COEFF / (num_tokens * num_tokens * TOP_K))


def kernel(x, W):
    xf = x.reshape(-1, x.shape[-1])
    num_tokens = xf.shape[0]
    nblk = num_tokens // (BLOCK_T * NSTREAM)
    stream_rows = num_tokens // NSTREAM

    def x_map(s):
        return lambda i: (i + s * nblk, 0)

    in_specs = [pl.BlockSpec((BLOCK_T, HIDDEN), x_map(s))
                for s in range(NSTREAM)]
    in_specs.append(pl.BlockSpec((NUM_EXPERTS, HIDDEN), lambda i: (0, 0)))

    out_specs = (
        [pl.BlockSpec((BLOCK_T, TOP_K), lambda i: (i, 0))
         for _ in range(2 * NSTREAM)]
        + [pl.BlockSpec((1, 1), lambda i: (0, 0))] * 2)
    out_shape = (
        [jax.ShapeDtypeStruct((stream_rows, TOP_K), jnp.float32)
         for _ in range(NSTREAM)]
        + [jax.ShapeDtypeStruct((stream_rows, TOP_K), jnp.int32)
           for _ in range(NSTREAM)]
        + [jax.ShapeDtypeStruct((1, 1), jnp.float32)] * 2)

    outs = pl.pallas_call(
        functools.partial(_router_block, nblk=nblk, num_tokens=num_tokens),
        grid=(nblk,),
        in_specs=in_specs,
        out_specs=out_specs,
        out_shape=out_shape,
        scratch_shapes=[
            pltpu.VMEM((1, NUM_EXPERTS), jnp.float32),
            pltpu.VMEM((1, NUM_EXPERTS), jnp.float32),
            pltpu.VMEM((1, 1), jnp.float32),
        ],
    )(*([xf] * NSTREAM), W)

    wts = jnp.concatenate(outs[:NSTREAM], axis=0)
    idx = jnp.concatenate(outs[NSTREAM:2 * NSTREAM], axis=0)
    z, lbl = outs[2 * NSTREAM], outs[2 * NSTREAM + 1]
    return wts, idx, z[0, 0], lbl[0, 0]


# speedup vs baseline: 1.0634x; 1.0092x over previous
import functools

import jax
import jax.numpy as jnp
from jax.experimental import pallas as pl
from jax.experimental.pallas import tpu as pltpu

NUM_EXPERTS = 64
TOP_K = 2
HIDDEN = 2048
Z_LOSS_COEFF = 0.001
AUX_LOSS_COEFF = 0.01

BLOCK_T = 1024
NSTREAM = 2
NEG_HUGE = -3.0e38


def _router_rows(logits):
    iota = jax.lax.broadcasted_iota(jnp.int32, logits.shape, 0)

    m1 = jnp.max(logits, axis=0, keepdims=True)
    idx1 = jnp.min(jnp.where(logits == m1, iota, NUM_EXPERTS),
                   axis=0, keepdims=True)
    masked = jnp.where(iota == idx1, NEG_HUGE, logits)
    m2 = jnp.max(masked, axis=0, keepdims=True)
    idx2 = jnp.min(jnp.where(masked == m2, iota, NUM_EXPERTS),
                   axis=0, keepdims=True)

    ex = jnp.exp(logits - m1)
    denom = jnp.sum(ex, axis=0, keepdims=True)
    recip = 1.0 / denom
    lse = m1 + jnp.log(denom)

    w1 = recip
    w2 = jnp.exp(m2 - m1) * recip

    wts_t = jnp.concatenate([w1, w2], axis=0)
    idx_t = jnp.concatenate([idx1, idx2], axis=0)

    onehot = ((iota == idx1) | (iota == idx2)).astype(jnp.float32)
    hist_part = jnp.sum(onehot, axis=1, keepdims=True)
    agg_part = jnp.sum(ex * recip, axis=1, keepdims=True)
    z_part = jnp.sum(lse * lse, axis=1, keepdims=True)
    return wts_t, idx_t, hist_part, agg_part, z_part


def _router_block(*refs, nblk, num_tokens):
    x_refs = refs[:NSTREAM]
    w_ref = refs[NSTREAM]
    wts_refs = refs[NSTREAM + 1:2 * NSTREAM + 1]
    idx_refs = refs[2 * NSTREAM + 1:3 * NSTREAM + 1]
    z_ref = refs[3 * NSTREAM + 1]
    lbl_ref = refs[3 * NSTREAM + 2]
    agg_ref, hist_ref, zacc_ref = refs[3 * NSTREAM + 3:]

    i = pl.program_id(0)
    w = w_ref[...]

    hist_acc = jnp.zeros((NUM_EXPERTS, 1), jnp.float32)
    agg_acc = jnp.zeros((NUM_EXPERTS, 1), jnp.float32)
    z_acc = jnp.zeros((1, 1), jnp.float32)
    for x_ref, wts_ref, idx_ref in zip(x_refs, wts_refs, idx_refs):
        logits = jax.lax.dot_general(
            w, x_ref[...],
            dimension_numbers=(((1,), (1,)), ((), ())),
            preferred_element_type=jnp.float32,
        )
        wts_t, idx_t, hist_part, agg_part, z_part = _router_rows(logits)
        wts_ref[...] = wts_t.T
        idx_ref[...] = idx_t.T
        hist_acc += hist_part
        agg_acc += agg_part
        z_acc += z_part

    @pl.when(i == 0)
    def _init():
        agg_ref[...] = agg_acc
        hist_ref[...] = hist_acc
        zacc_ref[...] = z_acc

    @pl.when(i > 0)
    def _accum():
        agg_ref[...] += agg_acc
        hist_ref[...] += hist_acc
        zacc_ref[...] += z_acc

    @pl.when(i == nblk - 1)
    def _finalize():
        z_ref[...] = zacc_ref[...] * (Z_LOSS_COEFF / num_tokens)
        lbl_ref[...] = jnp.sum(agg_ref[...] * hist_ref[...], keepdims=True) * (
            NUM_EXPERTS * AUX_LOSS_COEFF / (num_tokens * num_tokens * TOP_K))


def kernel(x, W):
    xf = x.reshape(-1, x.shape[-1])
    num_tokens = xf.shape[0]
    nblk = num_tokens // (BLOCK_T * NSTREAM)
    stream_rows = num_tokens // NSTREAM

    def x_map(s):
        return lambda i: (i + s * nblk, 0)

    in_specs = [pl.BlockSpec((BLOCK_T, HIDDEN), x_map(s))
                for s in range(NSTREAM)]
    in_specs.append(pl.BlockSpec((NUM_EXPERTS, HIDDEN), lambda i: (0, 0)))

    out_specs = (
        [pl.BlockSpec((BLOCK_T, TOP_K), lambda i: (i, 0))
         for _ in range(2 * NSTREAM)]
        + [pl.BlockSpec((1, 1), lambda i: (0, 0))] * 2)
    out_shape = (
        [jax.ShapeDtypeStruct((stream_rows, TOP_K), jnp.float32)
         for _ in range(NSTREAM)]
        + [jax.ShapeDtypeStruct((stream_rows, TOP_K), jnp.int32)
           for _ in range(NSTREAM)]
        + [jax.ShapeDtypeStruct((1, 1), jnp.float32)] * 2)

    outs = pl.pallas_call(
        functools.partial(_router_block, nblk=nblk, num_tokens=num_tokens),
        grid=(nblk,),
        in_specs=in_specs,
        out_specs=out_specs,
        out_shape=out_shape,
        scratch_shapes=[
            pltpu.VMEM((NUM_EXPERTS, 1), jnp.float32),
            pltpu.VMEM((NUM_EXPERTS, 1), jnp.float32),
            pltpu.VMEM((1, 1), jnp.float32),
        ],
    )(*([xf] * NSTREAM), W)

    wts = jnp.concatenate(outs[:NSTREAM], axis=0)
    idx = jnp.concatenate(outs[NSTREAM:2 * NSTREAM], axis=0)
    z, lbl = outs[2 * NSTREAM], outs[2 * NSTREAM + 1]
    return wts, idx, z[0, 0], lbl[0, 0]


# speedup vs baseline: 1.1270x; 1.0598x over previous
import functools

import jax
import jax.numpy as jnp
from jax.experimental import pallas as pl
from jax.experimental.pallas import tpu as pltpu

NUM_EXPERTS = 64
TOP_K = 2
HIDDEN = 2048
Z_LOSS_COEFF = 0.001
AUX_LOSS_COEFF = 0.01

BLOCK_T = 1024
NSTREAM = 2
NEG_HUGE = -3.0e38


def _router_rows(logits):
    iota = jax.lax.broadcasted_iota(jnp.int32, logits.shape, 0)

    m1 = jnp.max(logits, axis=0, keepdims=True)
    idx1 = jnp.min(jnp.where(logits == m1, iota, NUM_EXPERTS),
                   axis=0, keepdims=True)
    masked = jnp.where(iota == idx1, NEG_HUGE, logits)
    m2 = jnp.max(masked, axis=0, keepdims=True)
    idx2 = jnp.min(jnp.where(masked == m2, iota, NUM_EXPERTS),
                   axis=0, keepdims=True)

    ex = jnp.exp(logits - m1)
    denom = jnp.sum(ex, axis=0, keepdims=True)
    recip = 1.0 / denom
    lse = m1 + jnp.log(denom)

    w1 = recip
    w2 = jnp.exp(m2 - m1) * recip

    wts_t = jnp.concatenate([w1, w2], axis=0)
    idx_t = jnp.concatenate([idx1, idx2], axis=0)

    onehot = ((iota == idx1) | (iota == idx2)).astype(jnp.float32)
    hist_part = jnp.sum(onehot, axis=1, keepdims=True)
    agg_part = jnp.sum(ex * recip, axis=1, keepdims=True)
    z_part = jnp.sum(lse * lse, axis=1, keepdims=True)
    return wts_t, idx_t, hist_part, agg_part, z_part


def _router_block(*refs, nblk, num_tokens):
    x_refs = refs[:NSTREAM]
    w_ref = refs[NSTREAM]
    wts_refs = refs[NSTREAM + 1:2 * NSTREAM + 1]
    idx_refs = refs[2 * NSTREAM + 1:3 * NSTREAM + 1]
    z_ref = refs[3 * NSTREAM + 1]
    lbl_ref = refs[3 * NSTREAM + 2]
    agg_ref, hist_ref, zacc_ref = refs[3 * NSTREAM + 3:]

    i = pl.program_id(0)
    w = w_ref[...]

    hist_acc = jnp.zeros((NUM_EXPERTS, 1), jnp.float32)
    agg_acc = jnp.zeros((NUM_EXPERTS, 1), jnp.float32)
    z_acc = jnp.zeros((1, 1), jnp.float32)
    for x_ref, wts_ref, idx_ref in zip(x_refs, wts_refs, idx_refs):
        logits = jax.lax.dot_general(
            w, x_ref[...],
            dimension_numbers=(((1,), (1,)), ((), ())),
            preferred_element_type=jnp.float32,
        )
        wts_t, idx_t, hist_part, agg_part, z_part = _router_rows(logits)
        wts_ref[...] = wts_t
        idx_ref[...] = idx_t
        hist_acc += hist_part
        agg_acc += agg_part
        z_acc += z_part

    @pl.when(i == 0)
    def _init():
        agg_ref[...] = agg_acc
        hist_ref[...] = hist_acc
        zacc_ref[...] = z_acc

    @pl.when(i > 0)
    def _accum():
        agg_ref[...] += agg_acc
        hist_ref[...] += hist_acc
        zacc_ref[...] += z_acc

    @pl.when(i == nblk - 1)
    def _finalize():
        z_ref[...] = zacc_ref[...] * (Z_LOSS_COEFF / num_tokens)
        lbl_ref[...] = jnp.sum(agg_ref[...] * hist_ref[...], keepdims=True) * (
            NUM_EXPERTS * AUX_LOSS_COEFF / (num_tokens * num_tokens * TOP_K))


def kernel(x, W):
    xf = x.reshape(-1, x.shape[-1])
    num_tokens = xf.shape[0]
    nblk = num_tokens // (BLOCK_T * NSTREAM)
    stream_rows = num_tokens // NSTREAM

    def x_map(s):
        return lambda i: (i + s * nblk, 0)

    in_specs = [pl.BlockSpec((BLOCK_T, HIDDEN), x_map(s))
                for s in range(NSTREAM)]
    in_specs.append(pl.BlockSpec((NUM_EXPERTS, HIDDEN), lambda i: (0, 0)))

    out_specs = (
        [pl.BlockSpec((TOP_K, BLOCK_T), lambda i: (0, i))
         for _ in range(2 * NSTREAM)]
        + [pl.BlockSpec((1, 1), lambda i: (0, 0))] * 2)
    out_shape = (
        [jax.ShapeDtypeStruct((TOP_K, stream_rows), jnp.float32)
         for _ in range(NSTREAM)]
        + [jax.ShapeDtypeStruct((TOP_K, stream_rows), jnp.int32)
           for _ in range(NSTREAM)]
        + [jax.ShapeDtypeStruct((1, 1), jnp.float32)] * 2)

    outs = pl.pallas_call(
        functools.partial(_router_block, nblk=nblk, num_tokens=num_tokens),
        grid=(nblk,),
        in_specs=in_specs,
        out_specs=out_specs,
        out_shape=out_shape,
        scratch_shapes=[
            pltpu.VMEM((NUM_EXPERTS, 1), jnp.float32),
            pltpu.VMEM((NUM_EXPERTS, 1), jnp.float32),
            pltpu.VMEM((1, 1), jnp.float32),
        ],
    )(*([xf] * NSTREAM), W)

    wts = jnp.concatenate(outs[:NSTREAM], axis=1).T
    idx = jnp.concatenate(outs[NSTREAM:2 * NSTREAM], axis=1).T
    z, lbl = outs[2 * NSTREAM], outs[2 * NSTREAM + 1]
    return wts, idx, z[0, 0], lbl[0, 0]


# speedup vs baseline: 1.1351x; 1.0071x over previous
import functools

import jax
import jax.numpy as jnp
from jax.experimental import pallas as pl
from jax.experimental.pallas import tpu as pltpu

NUM_EXPERTS = 64
TOP_K = 2
HIDDEN = 2048
Z_LOSS_COEFF = 0.001
AUX_LOSS_COEFF = 0.01

BLOCK_T = 512
NSTREAM = 2
NEG_HUGE = -3.0e38


def _router_rows(logits):
    iota = jax.lax.broadcasted_iota(jnp.int32, logits.shape, 0)

    m1 = jnp.max(logits, axis=0, keepdims=True)
    idx1 = jnp.min(jnp.where(logits == m1, iota, NUM_EXPERTS),
                   axis=0, keepdims=True)
    masked = jnp.where(iota == idx1, NEG_HUGE, logits)
    m2 = jnp.max(masked, axis=0, keepdims=True)
    idx2 = jnp.min(jnp.where(masked == m2, iota, NUM_EXPERTS),
                   axis=0, keepdims=True)

    ex = jnp.exp(logits - m1)
    denom = jnp.sum(ex, axis=0, keepdims=True)
    recip = 1.0 / denom
    lse = m1 + jnp.log(denom)

    w1 = recip
    w2 = jnp.exp(m2 - m1) * recip

    wts_t = jnp.concatenate([w1, w2], axis=0)
    idx_t = jnp.concatenate([idx1, idx2], axis=0)

    onehot = ((iota == idx1) | (iota == idx2)).astype(jnp.float32)
    hist_part = jnp.sum(onehot, axis=1, keepdims=True)
    agg_part = jnp.sum(ex * recip, axis=1, keepdims=True)
    z_part = jnp.sum(lse * lse, axis=1, keepdims=True)
    return wts_t, idx_t, hist_part, agg_part, z_part


def _router_block(*refs, nblk, num_tokens):
    x_refs = refs[:NSTREAM]
    w_ref = refs[NSTREAM]
    wts_refs = refs[NSTREAM + 1:2 * NSTREAM + 1]
    idx_refs = refs[2 * NSTREAM + 1:3 * NSTREAM + 1]
    z_ref = refs[3 * NSTREAM + 1]
    lbl_ref = refs[3 * NSTREAM + 2]
    agg_ref, hist_ref, zacc_ref = refs[3 * NSTREAM + 3:]

    i = pl.program_id(0)
    w = w_ref[...]

    hist_acc = jnp.zeros((NUM_EXPERTS, 1), jnp.float32)
    agg_acc = jnp.zeros((NUM_EXPERTS, 1), jnp.float32)
    z_acc = jnp.zeros((1, 1), jnp.float32)
    for x_ref, wts_ref, idx_ref in zip(x_refs, wts_refs, idx_refs):
        logits = jax.lax.dot_general(
            w, x_ref[...],
            dimension_numbers=(((1,), (1,)), ((), ())),
            preferred_element_type=jnp.float32,
        )
        wts_t, idx_t, hist_part, agg_part, z_part = _router_rows(logits)
        wts_ref[...] = wts_t
        idx_ref[...] = idx_t
        hist_acc += hist_part
        agg_acc += agg_part
        z_acc += z_part

    @pl.when(i == 0)
    def _init():
        agg_ref[...] = agg_acc
        hist_ref[...] = hist_acc
        zacc_ref[...] = z_acc

    @pl.when(i > 0)
    def _accum():
        agg_ref[...] += agg_acc
        hist_ref[...] += hist_acc
        zacc_ref[...] += z_acc

    @pl.when(i == nblk - 1)
    def _finalize():
        z_ref[...] = zacc_ref[...] * (Z_LOSS_COEFF / num_tokens)
        lbl_ref[...] = jnp.sum(agg_ref[...] * hist_ref[...], keepdims=True) * (
            NUM_EXPERTS * AUX_LOSS_COEFF / (num_tokens * num_tokens * TOP_K))


def kernel(x, W):
    xf = x.reshape(-1, x.shape[-1])
    num_tokens = xf.shape[0]
    nblk = num_tokens // (BLOCK_T * NSTREAM)
    stream_rows = num_tokens // NSTREAM

    def x_map(s):
        return lambda i: (i + s * nblk, 0)

    in_specs = [pl.BlockSpec((BLOCK_T, HIDDEN), x_map(s))
                for s in range(NSTREAM)]
    in_specs.append(pl.BlockSpec((NUM_EXPERTS, HIDDEN), lambda i: (0, 0)))

    out_specs = (
        [pl.BlockSpec((TOP_K, BLOCK_T), lambda i: (0, i))
         for _ in range(2 * NSTREAM)]
        + [pl.BlockSpec((1, 1), lambda i: (0, 0))] * 2)
    out_shape = (
        [jax.ShapeDtypeStruct((TOP_K, stream_rows), jnp.float32)
         for _ in range(NSTREAM)]
        + [jax.ShapeDtypeStruct((TOP_K, stream_rows), jnp.int32)
           for _ in range(NSTREAM)]
        + [jax.ShapeDtypeStruct((1, 1), jnp.float32)] * 2)

    outs = pl.pallas_call(
        functools.partial(_router_block, nblk=nblk, num_tokens=num_tokens),
        grid=(nblk,),
        in_specs=in_specs,
        out_specs=out_specs,
        out_shape=out_shape,
        scratch_shapes=[
            pltpu.VMEM((NUM_EXPERTS, 1), jnp.float32),
            pltpu.VMEM((NUM_EXPERTS, 1), jnp.float32),
            pltpu.VMEM((1, 1), jnp.float32),
        ],
    )(*([xf] * NSTREAM), W)

    wts = jnp.concatenate(outs[:NSTREAM], axis=1).T
    idx = jnp.concatenate(outs[NSTREAM:2 * NSTREAM], axis=1).T
    z, lbl = outs[2 * NSTREAM], outs[2 * NSTREAM + 1]
    return wts, idx, z[0, 0], lbl[0, 0]
